# baseline (speedup 1.0000x reference)
import functools

import jax
import jax.numpy as jnp
from jax import lax
from jax.experimental import pallas as pl
from jax.experimental.pallas import tpu as pltpu
from jax.experimental.pallas import tpu_sc as plsc

B, L = 16384, 200
PAD = -1.0

NC, NS, LANES = 2, 16, 16
NW = NC * NS
ROWS_PER_W = B // NW
CHUNK_ROWS = 128
NCHUNK = ROWS_PER_W // CHUNK_ROWS
WORDS = CHUNK_ROWS * L
GROUPS = CHUNK_ROWS // LANES
SEARCH_ITERS = 8

_mesh = plsc.VectorSubcoreMesh(core_axis_name="c", subcore_axis_name="s")


@functools.partial(
    pl.kernel,
    out_type=[
        jax.ShapeDtypeStruct((B * L,), jnp.float32),
        jax.ShapeDtypeStruct((B,), jnp.int32),
    ],
    mesh=_mesh,
    scratch_types=[
        pltpu.VMEM((WORDS,), jnp.float32),
        pltpu.VMEM((ROWS_PER_W,), jnp.int32),
    ],
    compiler_params=pltpu.CompilerParams(needs_layout_passes=False),
)
def _dense_to_ragged(x_hbm, vals_hbm, len_hbm, buf, len_v):
    wid = lax.axis_index("s") * NC + lax.axis_index("c")
    base_word = wid * ROWS_PER_W * L
    iota16 = lax.iota(jnp.int32, 16)

    for chunk in range(NCHUNK):
        off = base_word + chunk * WORDS
        pltpu.sync_copy(x_hbm.at[pl.ds(off, WORDS)], buf)

        def grp(g, _):
            rowbase = (g * LANES + iota16) * L
            lo = jnp.zeros((16,), jnp.int32)
            hi = jnp.full((16,), L, jnp.int32)
            for _i in range(SEARCH_ITERS):
                active = lo < hi
                mid = jnp.right_shift(lo + hi, 1)
                midc = jnp.minimum(mid, L - 1)
                v = plsc.load_gather(buf, [rowbase + midc])
                is_pad = v == PAD
                lo2 = jnp.where(is_pad, lo, mid + 1)
                hi2 = jnp.where(is_pad, mid, hi)
                lo = jnp.where(active, lo2, lo)
                hi = jnp.where(active, hi2, hi)
            len_v[pl.ds(chunk * CHUNK_ROWS + g * LANES, 16)] = lo
            return 0

        lax.fori_loop(0, GROUPS, grp, 0)

        def ew(i, _):
            o = i * 16
            v = buf[pl.ds(o, 16)]
            buf[pl.ds(o, 16)] = jnp.where(v == PAD, jnp.float32(0.0), v)
            return 0

        lax.fori_loop(0, WORDS // 16, ew, 0, unroll=8)

        pltpu.sync_copy(buf, vals_hbm.at[pl.ds(off, WORDS)])

    pltpu.sync_copy(len_v, len_hbm.at[pl.ds(wid * ROWS_PER_W, ROWS_PER_W)])


def kernel(inputs):
    vals_flat, row_lengths = _dense_to_ragged(inputs.reshape(B * L))
    return vals_flat.reshape(B, L), row_lengths


# speedup vs baseline: 1.6890x; 1.6890x over previous
import functools

import jax
import jax.numpy as jnp
from jax import lax
from jax.experimental import pallas as pl
from jax.experimental.pallas import tpu as pltpu
from jax.experimental.pallas import tpu_sc as plsc

B, L = 16384, 200
PAD = -1.0

NC, NS, LANES = 2, 16, 16
NW = NC * NS
ROWS_PER_W = B // NW
CHUNK_ROWS = 128
NCHUNK = ROWS_PER_W // CHUNK_ROWS
GROUPS = CHUNK_ROWS // LANES
SEARCH_ITERS = 8

SLICE_OFFS = tuple(range(0, L - 16, 16)) + (L - 16,)

_mesh = plsc.VectorSubcoreMesh(core_axis_name="c", subcore_axis_name="s")


@functools.partial(
    pl.kernel,
    out_type=[
        jax.ShapeDtypeStruct((B, L), jnp.float32),
        jax.ShapeDtypeStruct((B,), jnp.int32),
    ],
    mesh=_mesh,
    scratch_types=[
        pltpu.VMEM((CHUNK_ROWS, L), jnp.float32),
        pltpu.VMEM((ROWS_PER_W,), jnp.int32),
    ],
    compiler_params=pltpu.CompilerParams(needs_layout_passes=False),
)
def _dense_to_ragged(x_hbm, vals_hbm, len_hbm, buf, len_v):
    wid = lax.axis_index("s") * NC + lax.axis_index("c")
    row_base = wid * ROWS_PER_W
    iota16 = lax.iota(jnp.int32, 16)

    for chunk in range(NCHUNK):
        r0 = row_base + chunk * CHUNK_ROWS
        pltpu.sync_copy(x_hbm.at[pl.ds(r0, CHUNK_ROWS)], buf)

        def grp(g, _):
            rows = g * LANES + iota16
            lo = jnp.zeros((16,), jnp.int32)
            hi = jnp.full((16,), L, jnp.int32)
            for _i in range(SEARCH_ITERS):
                active = lo < hi
                mid = jnp.right_shift(lo + hi, 1)
                midc = jnp.minimum(mid, L - 1)
                v = plsc.load_gather(buf, [rows, midc])
                is_pad = v == PAD
                lo2 = jnp.where(is_pad, lo, mid + 1)
                hi2 = jnp.where(is_pad, mid, hi)
                lo = jnp.where(active, lo2, lo)
                hi = jnp.where(active, hi2, hi)
            len_v[pl.ds(chunk * CHUNK_ROWS + g * LANES, 16)] = lo
            return 0

        lax.fori_loop(0, GROUPS, grp, 0)

        def ew(r, _):
            for o in SLICE_OFFS:
                v = buf[r, pl.ds(o, 16)]
                buf[r, pl.ds(o, 16)] = jnp.where(v == PAD, jnp.float32(0.0), v)
            return 0

        lax.fori_loop(0, CHUNK_ROWS, ew, 0)

        pltpu.sync_copy(buf, vals_hbm.at[pl.ds(r0, CHUNK_ROWS)])

    pltpu.sync_copy(len_v, len_hbm.at[pl.ds(wid * ROWS_PER_W, ROWS_PER_W)])


def kernel(inputs):
    values, row_lengths = _dense_to_ragged(inputs)
    return values, row_lengths
